# Initial kernel scaffold; baseline (speedup 1.0000x reference)
import jax
import jax.numpy as jnp
from jax.experimental import pallas as pl


def kernel(x, weight, gate_w, gate_b):
    raise NotImplementedError("write your pallas kernel here")


# speedup vs baseline: 4.5848x; 4.5848x over previous
import jax
import jax.numpy as jnp
from jax.experimental import pallas as pl

D_MODEL = 1024
E = 64
B = 64


def _gating_kernel(x_ref, gw_ref, gb_ref, g_out_ref):
    x = x_ref[...]
    gw = gw_ref[...]
    logits = jax.lax.dot_general(
        x, gw, (((1,), (1,)), ((), ())), preferred_element_type=jnp.float32
    ) + gb_ref[...]
    z = logits - jnp.max(logits, axis=1, keepdims=True)
    ez = jnp.exp(z)
    g = ez / jnp.sum(ez, axis=1, keepdims=True)
    m1 = jnp.max(g, axis=1, keepdims=True)
    g_wo_top1 = jnp.where(g == m1, -1.0, g)
    m2 = jnp.max(g_wo_top1, axis=1, keepdims=True)
    sel = g >= m2
    active = jnp.max(sel.astype(jnp.float32), axis=0, keepdims=True)
    g_out_ref[...] = g * active


def _expert_mm_kernel(x_ref, g_ref, w_ref, o_ref):
    e = pl.program_id(0)
    part = jnp.dot(x_ref[...], w_ref[...], preferred_element_type=jnp.float32)
    onehot = (jax.lax.broadcasted_iota(jnp.int32, (E, 1), 0) == e).astype(jnp.float32)
    col = jnp.dot(g_ref[...], onehot, preferred_element_type=jnp.float32)
    contrib = part * col

    @pl.when(e == 0)
    def _():
        o_ref[...] = contrib

    @pl.when(e > 0)
    def _():
        o_ref[...] += contrib


def kernel(x, weight, gate_w, gate_b):
    gb2 = gate_b.reshape(1, E)

    g_eff = pl.pallas_call(
        _gating_kernel,
        out_shape=jax.ShapeDtypeStruct((B, E), jnp.float32),
    )(x, gate_w, gb2)

    out = pl.pallas_call(
        _expert_mm_kernel,
        grid=(E,),
        in_specs=[
            pl.BlockSpec((B, D_MODEL), lambda e: (0, 0)),
            pl.BlockSpec((B, E), lambda e: (0, 0)),
            pl.BlockSpec((D_MODEL, D_MODEL), lambda e: (0, e)),
        ],
        out_specs=pl.BlockSpec((B, D_MODEL), lambda e: (0, 0)),
        out_shape=jax.ShapeDtypeStruct((B, D_MODEL), jnp.float32),
    )(x, g_eff, weight)
    return out
